# baseline (speedup 1.0000x reference)
import jax
import jax.numpy as jnp
from jax.experimental import pallas as pl
from jax.experimental.pallas import tpu as pltpu

_NUM_FIELDS = 26
_DEPTH = 1000
_BR = 64
_NBUF = 4


def _onehot_block(fv_ref, out_ref, scratch, sems):
    g = pl.program_id(0)
    nsteps = pl.num_programs(0)
    iota = jax.lax.broadcasted_iota(jnp.int32, (_BR, _DEPTH), 1)
    for b in range(_NBUF):
        @pl.when(g >= 1)
        def _wait_prev(b=b):
            prev0 = ((g - 1) * _NBUF + b) * _BR
            pltpu.make_async_copy(
                scratch.at[b], out_ref.at[pl.ds(prev0, _BR), :],
                sems.at[b]).wait()

        row0 = (g * _NBUF + b) * _BR
        fv = fv_ref[pl.ds(row0, _BR), :]
        for f in range(_NUM_FIELDS):
            scratch[b, :, f * _DEPTH:(f + 1) * _DEPTH] = (
                iota == fv[:, f:f + 1]).astype(jnp.float32)
        pltpu.make_async_copy(
            scratch.at[b], out_ref.at[pl.ds(row0, _BR), :],
            sems.at[b]).start()

    @pl.when(g == nsteps - 1)
    def _drain():
        for b in range(_NBUF):
            row0 = (g * _NBUF + b) * _BR
            pltpu.make_async_copy(
                scratch.at[b], out_ref.at[pl.ds(row0, _BR), :],
                sems.at[b]).wait()


def kernel(feature_value):
    batch = feature_value.shape[0]
    width = _NUM_FIELDS * _DEPTH
    return pl.pallas_call(
        _onehot_block,
        grid=(batch // (_BR * _NBUF),),
        in_specs=[pl.BlockSpec(memory_space=pltpu.MemorySpace.VMEM)],
        out_specs=pl.BlockSpec(memory_space=pl.ANY),
        out_shape=jax.ShapeDtypeStruct((batch, width), jnp.float32),
        scratch_shapes=[
            pltpu.VMEM((_NBUF, _BR, width), jnp.float32),
            pltpu.SemaphoreType.DMA((_NBUF,)),
        ],
        compiler_params=pltpu.CompilerParams(
            dimension_semantics=("arbitrary",)),
    )(feature_value)


# speedup vs baseline: 4.0809x; 4.0809x over previous
import jax
import jax.numpy as jnp
from jax.experimental import pallas as pl

_NUM_FIELDS = 26
_DEPTH = 1000


def _onehot_t_block(fvt_ref, out_ref):
    fv_row = fvt_ref[0]
    pos = jax.lax.broadcasted_iota(jnp.int32, out_ref.shape, 0)
    out_ref[...] = (pos == fv_row).astype(jnp.float32)


def kernel(feature_value):
    batch = feature_value.shape[0]
    fvt = feature_value.T.reshape(_NUM_FIELDS, 1, batch)
    out_t = pl.pallas_call(
        _onehot_t_block,
        grid=(_NUM_FIELDS,),
        in_specs=[pl.BlockSpec((1, 1, batch), lambda f: (f, 0, 0))],
        out_specs=pl.BlockSpec((_DEPTH, batch), lambda f: (f, 0)),
        out_shape=jax.ShapeDtypeStruct((_NUM_FIELDS * _DEPTH, batch),
                                       jnp.float32),
    )(fvt)
    return out_t.T
